# Initial kernel scaffold; baseline (speedup 1.0000x reference)
import jax
import jax.numpy as jnp
from jax.experimental import pallas as pl


def kernel(x, edge_index, start_node, target_node, W1l, W1r, b1, W2l, W2r, b2, W3l, W3r, b3, W4l, W4r, b4, fcW, fcb):
    raise NotImplementedError("write your pallas kernel here")


# speedup vs baseline: 4.1121x; 4.1121x over previous
import functools

import jax
import jax.numpy as jnp
from jax import lax
from jax.experimental import pallas as pl
from jax.experimental.pallas import tpu as pltpu
from jax.experimental.pallas import tpu_sc as plsc

N = 10000
E = 640000
H = 128
OUTDIM = N + 1
D1 = 128

NC, NS, L = 2, 16, 16
NW = NC * NS
CH = 128
SB = 16
NSB = 10
G = SB * NSB
EPW = G * CH
ACC_ROWS = N + 16
RPS = 624
ZR = 8


def _make_sc_segment_sum(D):
  mesh = plsc.VectorSubcoreMesh(core_axis_name="c", subcore_axis_name="s")

  @functools.partial(
      pl.kernel,
      mesh=mesh,
      out_type=jax.ShapeDtypeStruct((NC, N, D), jnp.float32),
      scratch_types=[
          pltpu.VMEM((SB, CH), jnp.int32),
          pltpu.VMEM((SB, CH), jnp.int32),
          pltpu.VMEM((CH, D), jnp.float32),
          pltpu.VMEM((CH, D), jnp.float32),
          pltpu.VMEM((ZR, D), jnp.float32),
          pltpu.VMEM_SHARED((ACC_ROWS, D), jnp.float32),
          pltpu.SemaphoreType.DMA,
          pltpu.SemaphoreType.DMA,
      ],
  )
  def k(h_hbm, srcs_hbm, dsts_hbm, out_hbm,
        src_v, dst_v, rows0, rows1, zbuf, acc, sem0, sem1):
    cid = lax.axis_index("c")
    sid = lax.axis_index("s")
    wid = sid * NC + cid

    zeros16 = jnp.zeros((L,), jnp.float32)
    for r in range(ZR):
      for j in range(D // L):
        zbuf[r, pl.ds(j * L, L)] = zeros16

    def zcopy(t, carry):
      pltpu.sync_copy(zbuf, acc.at[pl.ds(sid * RPS + t * ZR, ZR)])
      return carry

    lax.fori_loop(0, RPS // ZR, zcopy, 0)

    @pl.when(sid == NS - 1)
    def _zero_tail():
      for t in range((ACC_ROWS - NS * RPS) // ZR):
        pltpu.sync_copy(zbuf, acc.at[pl.ds(NS * RPS + t * ZR, ZR)])

    plsc.subcore_barrier()

    bufs = ((rows0, sem0), (rows1, sem1))

    def superblock(s, carry):
      pltpu.sync_copy(srcs_hbm.at[wid, pl.ds(s * SB, SB)], src_v)
      pltpu.sync_copy(dsts_hbm.at[wid, pl.ds(s * SB, SB)], dst_v)
      pltpu.async_copy(h_hbm.at[src_v.at[0]], rows0, sem0)
      pltpu.async_copy(h_hbm.at[src_v.at[1]], rows1, sem1)
      for i in range(SB):
        rows, sem = bufs[i % 2]
        pltpu.make_async_copy(h_hbm.at[pl.ds(0, CH)], rows, sem).wait()
        pltpu.sync_copy(rows, acc.at[dst_v.at[i]], add=True)
        if i + 2 < SB:
          pltpu.async_copy(h_hbm.at[src_v.at[i + 2]], rows, sem)
      return carry

    lax.fori_loop(0, NSB, superblock, 0)

    plsc.subcore_barrier()
    pltpu.sync_copy(acc.at[pl.ds(sid * RPS, RPS)],
                    out_hbm.at[cid, pl.ds(sid * RPS, RPS)])

    @pl.when(sid == NS - 1)
    def _out_tail():
      pltpu.sync_copy(acc.at[pl.ds(NS * RPS, N - NS * RPS)],
                      out_hbm.at[cid, pl.ds(NS * RPS, N - NS * RPS)])

  return k


_sc_segsum_128 = _make_sc_segment_sum(H)


def _l1_body(p_ref, h_ref, wl_ref, wr_ref, b_ref, out_ref, cnt_ref):
  ps = p_ref[0] + p_ref[1]
  cnt = jnp.maximum(ps[:, 34], 1.0)
  mean = ps / cnt[:, None]
  acc = lax.dot_general(mean, wl_ref[...], (((1,), (1,)), ((), ())),
                        preferred_element_type=jnp.float32)
  acc += lax.dot_general(h_ref[...], wr_ref[...], (((1,), (1,)), ((), ())),
                         preferred_element_type=jnp.float32)
  out_ref[...] = jnp.maximum(acc + b_ref[...][None, :], 0.0)
  cnt_ref[...] = cnt


def _lx_body(p_ref, h_ref, cnt_ref, wl_ref, wr_ref, b_ref, out_ref):
  ps = p_ref[0] + p_ref[1]
  mean = ps / cnt_ref[...][:, None]
  acc = lax.dot_general(mean, wl_ref[...], (((1,), (1,)), ((), ())),
                        preferred_element_type=jnp.float32)
  acc += lax.dot_general(h_ref[...], wr_ref[...], (((1,), (1,)), ((), ())),
                         preferred_element_type=jnp.float32)
  out_ref[...] = jnp.maximum(acc + b_ref[...][None, :], 0.0)


def _fc_body(h_ref, w_ref, b_ref, out_ref):
  acc = lax.dot_general(h_ref[...], w_ref[...], (((1,), (1,)), ((), ())),
                        preferred_element_type=jnp.float32)
  out_ref[...] = acc + b_ref[...][None, :]


_BN = 2048


def _layer1(p, h0, wl, wr, b):
  grid = (pl.cdiv(N, _BN),)
  return pl.pallas_call(
      _l1_body,
      grid=grid,
      in_specs=[
          pl.BlockSpec((NC, _BN, D1), lambda i: (0, i, 0)),
          pl.BlockSpec((_BN, D1), lambda i: (i, 0)),
          pl.BlockSpec((H, D1), lambda i: (0, 0)),
          pl.BlockSpec((H, D1), lambda i: (0, 0)),
          pl.BlockSpec((H,), lambda i: (0,)),
      ],
      out_specs=[
          pl.BlockSpec((_BN, H), lambda i: (i, 0)),
          pl.BlockSpec((_BN,), lambda i: (i,)),
      ],
      out_shape=[
          jax.ShapeDtypeStruct((N, H), jnp.float32),
          jax.ShapeDtypeStruct((N,), jnp.float32),
      ],
  )(p, h0, wl, wr, b)


def _layerx(p, h, cnt, wl, wr, b):
  grid = (pl.cdiv(N, _BN),)
  return pl.pallas_call(
      _lx_body,
      grid=grid,
      in_specs=[
          pl.BlockSpec((NC, _BN, H), lambda i: (0, i, 0)),
          pl.BlockSpec((_BN, H), lambda i: (i, 0)),
          pl.BlockSpec((_BN,), lambda i: (i,)),
          pl.BlockSpec((H, H), lambda i: (0, 0)),
          pl.BlockSpec((H, H), lambda i: (0, 0)),
          pl.BlockSpec((H,), lambda i: (0,)),
      ],
      out_specs=pl.BlockSpec((_BN, H), lambda i: (i, 0)),
      out_shape=jax.ShapeDtypeStruct((N, H), jnp.float32),
  )(p, h, cnt, wl, wr, b)


_FBN = 1000
_FBC = 1024


def _fc(h, w, b):
  grid = (N // _FBN, pl.cdiv(OUTDIM, _FBC))
  return pl.pallas_call(
      _fc_body,
      grid=grid,
      in_specs=[
          pl.BlockSpec((_FBN, H), lambda i, j: (i, 0)),
          pl.BlockSpec((_FBC, H), lambda i, j: (j, 0)),
          pl.BlockSpec((_FBC,), lambda i, j: (j,)),
      ],
      out_specs=pl.BlockSpec((_FBN, _FBC), lambda i, j: (i, j)),
      out_shape=jax.ShapeDtypeStruct((N, OUTDIM), jnp.float32),
  )(h, w, b)


def kernel(x, edge_index, start_node, target_node,
           W1l, W1r, b1, W2l, W2r, b2, W3l, W3r, b3, W4l, W4r, b4, fcW, fcb):
  src = edge_index[0]
  dst = edge_index[1]

  pad = NW * EPW - E
  srcs = jnp.concatenate([src, jnp.zeros((pad,), jnp.int32)]).reshape(NW, G, CH)
  dsts = jnp.concatenate([dst, jnp.full((pad,), N, jnp.int32)]).reshape(NW, G, CH)

  h0 = jnp.zeros((N, D1), jnp.float32)
  h0 = h0.at[:, :32].set(x)
  h0 = h0.at[start_node, 32].set(1.0)
  h0 = h0.at[target_node, 33].set(1.0)
  h0 = h0.at[:, 34].set(1.0)

  wl1 = jnp.pad(W1l, ((0, 0), (0, D1 - 34)))
  wr1 = jnp.pad(W1r, ((0, 0), (0, D1 - 34)))

  p1 = _sc_segsum_128(h0, srcs, dsts)
  h1, cnt = _layer1(p1, h0, wl1, wr1, b1)

  p2 = _sc_segsum_128(h1, srcs, dsts)
  h2 = _layerx(p2, h1, cnt, W2l, W2r, b2)

  p3 = _sc_segsum_128(h2, srcs, dsts)
  h3 = _layerx(p3, h2, cnt, W3l, W3r, b3)

  p4 = _sc_segsum_128(h3, srcs, dsts)
  h4 = _layerx(p4, h3, cnt, W4l, W4r, b4)

  return _fc(h4, fcW, fcb)
